# Initial kernel scaffold; baseline (speedup 1.0000x reference)
import jax
import jax.numpy as jnp
from jax.experimental import pallas as pl


def kernel(x, params, edge_index, batch):
    raise NotImplementedError("write your pallas kernel here")


# speedup vs baseline: 3.2725x; 3.2725x over previous
import functools

import jax
import jax.numpy as jnp
from jax import lax
from jax.experimental import pallas as pl
from jax.experimental.pallas import tpu as pltpu
from jax.experimental.pallas import tpu_sc as plsc

BLK = 256
NGRAPH = 64
CH = 128
NTILES = 32


@functools.lru_cache(maxsize=None)
def _make_sc_agg(npad, d, ept):
    rows_per_tile = npad // 16
    nch = ept // CH
    mesh = plsc.VectorSubcoreMesh(core_axis_name="c", subcore_axis_name="s")

    @functools.partial(
        pl.kernel,
        out_type=jax.ShapeDtypeStruct((2 * npad, d), jnp.float32),
        mesh=mesh,
        scratch_types=[
            pltpu.VMEM((CH,), jnp.int32),
            pltpu.VMEM((CH,), jnp.int32),
            pltpu.VMEM((CH, d), jnp.float32),
            pltpu.VMEM_SHARED((npad, d), jnp.float32),
            pltpu.SemaphoreType.DMA,
        ],
    )
    def sc_agg(g_hbm, src_hbm, dst_hbm, out_hbm, src_v, dst_v, rows_v, acc_sh, sem):
        c = lax.axis_index("c")
        s = lax.axis_index("s")
        tid = s * 2 + c
        pltpu.sync_copy(
            g_hbm.at[pl.ds(s * rows_per_tile, rows_per_tile)],
            acc_sh.at[pl.ds(s * rows_per_tile, rows_per_tile)],
        )
        plsc.subcore_barrier()

        def body(i, carry):
            base = tid * ept + i * CH
            pltpu.sync_copy(src_hbm.at[pl.ds(base, CH)], src_v)
            pltpu.sync_copy(dst_hbm.at[pl.ds(base, CH)], dst_v)
            pltpu.async_copy(g_hbm.at[src_v], rows_v, sem).wait()
            pltpu.sync_copy(rows_v, acc_sh.at[dst_v], add=True)
            return carry

        lax.fori_loop(0, nch, body, 0)
        plsc.subcore_barrier()
        pltpu.sync_copy(
            acc_sh.at[pl.ds(s * rows_per_tile, rows_per_tile)],
            out_hbm.at[pl.ds(c * npad + s * rows_per_tile, rows_per_tile)],
        )

    return sc_agg


def _mlp(hin, w1, b1, g1, be1, w2, b2):
    a = jnp.dot(hin, w1, preferred_element_type=jnp.float32) + b1
    a = jnp.maximum(a * g1 + be1, 0.0)
    return jnp.dot(a, w2, preferred_element_type=jnp.float32) + b2


def _onehot(b_ref):
    bidx = b_ref[0, 0, :].reshape(BLK, 1)
    io = lax.broadcasted_iota(jnp.int32, (BLK, NGRAPH), 1)
    return (bidx == io).astype(jnp.float32)


def _full(shape):
    nd = len(shape)
    return pl.BlockSpec(shape, lambda i: (0,) * nd)


def _rows(d):
    return pl.BlockSpec((BLK, d), lambda i: (i, 0))


@functools.lru_cache(maxsize=None)
def _make_tc1(npad, d):
    nb = npad // BLK
    p1off = npad // BLK

    def body(p_ref, q_ref, g_ref, w1, b1, g1, be1, w2, b2, bg, bb, ve, o_ref):
        hin = p_ref[...] + q_ref[...] - g_ref[...]
        h = _mlp(hin, w1[...], b1[...], g1[...], be1[...], w2[...], b2[...])
        h = jnp.maximum(h * bg[...] + bb[...], 0.0)
        o_ref[...] = h + ve[...]

    return pl.pallas_call(
        body,
        grid=(nb,),
        in_specs=[
            _rows(d),
            pl.BlockSpec((BLK, d), lambda i: (p1off + i, 0)),
            _rows(d),
            _full((d, 2 * d)), _full((1, 2 * d)), _full((1, 2 * d)), _full((1, 2 * d)),
            _full((2 * d, d)), _full((1, d)),
            _full((1, d)), _full((1, d)), _full((1, d)),
        ],
        out_specs=_rows(d),
        out_shape=jax.ShapeDtypeStruct((npad, d), jnp.float32),
    )


@functools.lru_cache(maxsize=None)
def _make_tc2(npad, d):
    nb = npad // BLK
    p1off = npad // BLK

    def body(p_ref, q_ref, g_ref, b_ref, w1, b1, g1, be1, w2, b2, bg, bb,
             post_ref, pool_ref):
        i = pl.program_id(0)
        hin = p_ref[...] + q_ref[...] - g_ref[...]
        h = _mlp(hin, w1[...], b1[...], g1[...], be1[...], w2[...], b2[...])
        h = jnp.maximum(h * bg[...] + bb[...], 0.0)
        post_ref[...] = h
        oh = _onehot(b_ref)

        @pl.when(i == 0)
        def _():
            pool_ref[...] = jnp.zeros_like(pool_ref)

        pool_ref[...] += lax.dot_general(
            oh, h, (((0,), (0,)), ((), ())), preferred_element_type=jnp.float32)

    return pl.pallas_call(
        body,
        grid=(nb,),
        in_specs=[
            _rows(d),
            pl.BlockSpec((BLK, d), lambda i: (p1off + i, 0)),
            _rows(d),
            pl.BlockSpec((1, 1, BLK), lambda i: (i, 0, 0)),
            _full((d, 2 * d)), _full((1, 2 * d)), _full((1, 2 * d)), _full((1, 2 * d)),
            _full((2 * d, d)), _full((1, d)),
            _full((1, d)), _full((1, d)),
        ],
        out_specs=[_rows(d), _full((NGRAPH, d))],
        out_shape=[
            jax.ShapeDtypeStruct((npad, d), jnp.float32),
            jax.ShapeDtypeStruct((NGRAPH, d), jnp.float32),
        ],
    )


@functools.lru_cache(maxsize=None)
def _make_tcv(npad, d):
    nb = npad // BLK

    def body(post_ref, b_ref, pool_ref, ve, wa, ba, ga, bea, wb, bb2, gb, beb,
             o_ref, vf_ref):
        i = pl.program_id(0)

        @pl.when(i == 0)
        def _():
            h = pool_ref[...] + ve[...]
            a = jnp.dot(h, wa[...], preferred_element_type=jnp.float32) + ba[...]
            a = jnp.maximum(a * ga[...] + bea[...], 0.0)
            v = jnp.dot(a, wb[...], preferred_element_type=jnp.float32) + bb2[...]
            vf_ref[...] = jnp.maximum(v * gb[...] + beb[...], 0.0)

        oh = _onehot(b_ref)
        o_ref[...] = post_ref[...] + jnp.dot(
            oh, vf_ref[...], preferred_element_type=jnp.float32)

    return pl.pallas_call(
        body,
        grid=(nb,),
        in_specs=[
            _rows(d),
            pl.BlockSpec((1, 1, BLK), lambda i: (i, 0, 0)),
            _full((NGRAPH, d)), _full((1, d)),
            _full((d, 2 * d)), _full((1, 2 * d)), _full((1, 2 * d)), _full((1, 2 * d)),
            _full((2 * d, d)), _full((1, d)), _full((1, d)), _full((1, d)),
        ],
        out_specs=_rows(d),
        out_shape=jax.ShapeDtypeStruct((npad, d), jnp.float32),
        scratch_shapes=[pltpu.VMEM((NGRAPH, d), jnp.float32)],
    )


@functools.lru_cache(maxsize=None)
def _make_tc3(npad, d, ncls):
    nb = npad // BLK
    p1off = npad // BLK

    def body(p_ref, q_ref, g_ref, b_ref, w1, b1, g1, be1, w2, b2, bg, bb,
             wc, bc, o_ref, sum_scr, cnt_scr):
        i = pl.program_id(0)
        hin = p_ref[...] + q_ref[...] - g_ref[...]
        h = _mlp(hin, w1[...], b1[...], g1[...], be1[...], w2[...], b2[...])
        h = h * bg[...] + bb[...]
        oh = _onehot(b_ref)

        @pl.when(i == 0)
        def _():
            sum_scr[...] = jnp.zeros_like(sum_scr)
            cnt_scr[...] = jnp.zeros_like(cnt_scr)

        sum_scr[...] += lax.dot_general(
            oh, h, (((0,), (0,)), ((), ())), preferred_element_type=jnp.float32)
        cnt_scr[...] += jnp.sum(oh, axis=0).reshape(NGRAPH, 1)

        @pl.when(i == nb - 1)
        def _():
            readout = sum_scr[...] / jnp.maximum(cnt_scr[...], 1.0)
            o_ref[...] = jnp.dot(
                readout, wc[...], preferred_element_type=jnp.float32) + bc[...]

    return pl.pallas_call(
        body,
        grid=(nb,),
        in_specs=[
            _rows(d),
            pl.BlockSpec((BLK, d), lambda i: (p1off + i, 0)),
            _rows(d),
            pl.BlockSpec((1, 1, BLK), lambda i: (i, 0, 0)),
            _full((d, 2 * d)), _full((1, 2 * d)), _full((1, 2 * d)), _full((1, 2 * d)),
            _full((2 * d, d)), _full((1, d)),
            _full((1, d)), _full((1, d)),
            _full((d, ncls)), _full((1, ncls)),
        ],
        out_specs=_full((NGRAPH, ncls)),
        out_shape=jax.ShapeDtypeStruct((NGRAPH, ncls), jnp.float32),
        scratch_shapes=[
            pltpu.VMEM((NGRAPH, d), jnp.float32),
            pltpu.VMEM((NGRAPH, 1), jnp.float32),
        ],
    )


def _r2(v, w):
    return v.reshape(1, w)


def kernel(x, params, edge_index, batch):
    n, d = x.shape
    e = edge_index.shape[1]
    ncls = params["cls"]["Wc"].shape[1]

    npad = ((n + BLK) // BLK) * BLK
    ept = -(-e // (NTILES * CH)) * CH
    epad = NTILES * ept

    x_pad = jnp.zeros((npad, d), jnp.float32).at[:n].set(x)
    src_pad = jnp.concatenate([edge_index[0], jnp.zeros((epad - e,), jnp.int32)])
    dst_pad = jnp.concatenate([edge_index[1], jnp.full((epad - e,), n, jnp.int32)])
    batch3 = jnp.concatenate(
        [batch, jnp.full((npad - n,), NGRAPH, jnp.int32)]).reshape(npad // BLK, 1, BLK)

    sc_agg = _make_sc_agg(npad, d, ept)
    c1 = params["conv1"]
    bn1 = params["bn1"]
    vm = params["vmlp"]
    ve = params["vemb"]

    p = sc_agg(x_pad, src_pad, dst_pad)
    g2 = _make_tc1(npad, d)(
        p, p, x_pad,
        c1["W1"], _r2(c1["b1"], 2 * d), _r2(c1["g1"], 2 * d), _r2(c1["be1"], 2 * d),
        c1["W2"], _r2(c1["b2"], d),
        _r2(bn1["g"], d), _r2(bn1["b"], d), ve,
    )

    c2 = params["convs"][0]
    bn2 = params["bns"][0]
    p = sc_agg(g2, src_pad, dst_pad)
    post2, pooled = _make_tc2(npad, d)(
        p, p, g2, batch3,
        c2["W1"], _r2(c2["b1"], 2 * d), _r2(c2["g1"], 2 * d), _r2(c2["be1"], 2 * d),
        c2["W2"], _r2(c2["b2"], d),
        _r2(bn2["g"], d), _r2(bn2["b"], d),
    )

    g3 = _make_tcv(npad, d)(
        post2, batch3, pooled, ve,
        vm["Wa"], _r2(vm["ba"], 2 * d), _r2(vm["ga"], 2 * d), _r2(vm["bea"], 2 * d),
        vm["Wb"], _r2(vm["bb"], d), _r2(vm["gb"], d), _r2(vm["beb"], d),
    )

    c3 = params["convs"][1]
    bn3 = params["bns"][1]
    p = sc_agg(g3, src_pad, dst_pad)
    logits = _make_tc3(npad, d, ncls)(
        p, p, g3, batch3,
        c3["W1"], _r2(c3["b1"], 2 * d), _r2(c3["g1"], 2 * d), _r2(c3["be1"], 2 * d),
        c3["W2"], _r2(c3["b2"], d),
        _r2(bn3["g"], d), _r2(bn3["b"], d),
        params["cls"]["Wc"], _r2(params["cls"]["bc"], ncls),
    )
    return logits


# speedup vs baseline: 3.5396x; 1.0816x over previous
import functools

import jax
import jax.numpy as jnp
from jax import lax
from jax.experimental import pallas as pl
from jax.experimental.pallas import tpu as pltpu
from jax.experimental.pallas import tpu_sc as plsc

BLK = 256
NGRAPH = 64
CH = 128
NTILES = 32


@functools.lru_cache(maxsize=None)
def _make_sc_agg(npad, d, ept):
    rows_per_tile = npad // 16
    nch = ept // CH
    assert nch % 2 == 0
    mesh = plsc.VectorSubcoreMesh(core_axis_name="c", subcore_axis_name="s")

    @functools.partial(
        pl.kernel,
        out_type=jax.ShapeDtypeStruct((2 * npad, d), jnp.float32),
        mesh=mesh,
        scratch_types=[
            pltpu.VMEM((ept,), jnp.int32),
            pltpu.VMEM((CH,), jnp.int32),
            pltpu.VMEM((CH,), jnp.int32),
            pltpu.VMEM((CH, d), jnp.float32),
            pltpu.VMEM((CH, d), jnp.float32),
            pltpu.VMEM_SHARED((npad, d), jnp.float32),
            pltpu.SemaphoreType.DMA,
            pltpu.SemaphoreType.DMA,
            pltpu.SemaphoreType.DMA,
            pltpu.SemaphoreType.DMA,
        ],
    )
    def sc_agg(g_hbm, src_hbm, dst_hbm, out_hbm, src_st, dst0, dst1,
               rows0, rows1, acc_sh, semg0, semg1, semi0, semi1):
        c = lax.axis_index("c")
        s = lax.axis_index("s")
        tid = s * 2 + c
        ebase = tid * ept
        pltpu.sync_copy(
            g_hbm.at[pl.ds(s * rows_per_tile, rows_per_tile)],
            acc_sh.at[pl.ds(s * rows_per_tile, rows_per_tile)],
        )
        pltpu.sync_copy(src_hbm.at[pl.ds(ebase, ept)], src_st)
        plsc.subcore_barrier()

        def fetch(j, dst_v, rows, semg, semi):
            pltpu.async_copy(dst_hbm.at[pl.ds(ebase + j * CH, CH)], dst_v, semi)
            pltpu.async_copy(g_hbm.at[src_st.at[pl.ds(j * CH, CH)]], rows, semg)

        def step(j, dst_v, rows, semg, semi, prefetch):
            pltpu.make_async_copy(
                g_hbm.at[src_st.at[pl.ds(j * CH, CH)]], rows, semg).wait()
            pltpu.make_async_copy(dst_hbm.at[pl.ds(0, CH)], dst_v, semi).wait()
            pltpu.sync_copy(rows, acc_sh.at[dst_v], add=True)
            if prefetch:
                fetch(j + 2, dst_v, rows, semg, semi)

        fetch(0, dst0, rows0, semg0, semi0)
        fetch(1, dst1, rows1, semg1, semi1)

        def body(i, carry):
            j = 2 * i
            step(j, dst0, rows0, semg0, semi0, True)
            step(j + 1, dst1, rows1, semg1, semi1, True)
            return carry

        lax.fori_loop(0, nch // 2 - 1, body, 0)
        step(nch - 2, dst0, rows0, semg0, semi0, False)
        step(nch - 1, dst1, rows1, semg1, semi1, False)

        plsc.subcore_barrier()
        pltpu.sync_copy(
            acc_sh.at[pl.ds(s * rows_per_tile, rows_per_tile)],
            out_hbm.at[pl.ds(c * npad + s * rows_per_tile, rows_per_tile)],
        )

    return sc_agg


def _mlp(hin, w1, b1, g1, be1, w2, b2):
    a = jnp.dot(hin, w1, preferred_element_type=jnp.float32) + b1
    a = jnp.maximum(a * g1 + be1, 0.0)
    return jnp.dot(a, w2, preferred_element_type=jnp.float32) + b2


def _onehot(b_ref):
    bidx = b_ref[0, 0, :].reshape(BLK, 1)
    io = lax.broadcasted_iota(jnp.int32, (BLK, NGRAPH), 1)
    return (bidx == io).astype(jnp.float32)


def _full(shape):
    nd = len(shape)
    return pl.BlockSpec(shape, lambda i: (0,) * nd)


def _rows(d):
    return pl.BlockSpec((BLK, d), lambda i: (i, 0))


@functools.lru_cache(maxsize=None)
def _make_tc1(npad, d):
    nb = npad // BLK
    p1off = npad // BLK

    def body(p_ref, q_ref, g_ref, w1, b1, g1, be1, w2, b2, bg, bb, ve, o_ref):
        hin = p_ref[...] + q_ref[...] - g_ref[...]
        h = _mlp(hin, w1[...], b1[...], g1[...], be1[...], w2[...], b2[...])
        h = jnp.maximum(h * bg[...] + bb[...], 0.0)
        o_ref[...] = h + ve[...]

    return pl.pallas_call(
        body,
        grid=(nb,),
        in_specs=[
            _rows(d),
            pl.BlockSpec((BLK, d), lambda i: (p1off + i, 0)),
            _rows(d),
            _full((d, 2 * d)), _full((1, 2 * d)), _full((1, 2 * d)), _full((1, 2 * d)),
            _full((2 * d, d)), _full((1, d)),
            _full((1, d)), _full((1, d)), _full((1, d)),
        ],
        out_specs=_rows(d),
        out_shape=jax.ShapeDtypeStruct((npad, d), jnp.float32),
    )


@functools.lru_cache(maxsize=None)
def _make_tc2(npad, d):
    nb = npad // BLK
    p1off = npad // BLK

    def body(p_ref, q_ref, g_ref, b_ref, w1, b1, g1, be1, w2, b2, bg, bb,
             post_ref, pool_ref):
        i = pl.program_id(0)
        hin = p_ref[...] + q_ref[...] - g_ref[...]
        h = _mlp(hin, w1[...], b1[...], g1[...], be1[...], w2[...], b2[...])
        h = jnp.maximum(h * bg[...] + bb[...], 0.0)
        post_ref[...] = h
        oh = _onehot(b_ref)

        @pl.when(i == 0)
        def _():
            pool_ref[...] = jnp.zeros_like(pool_ref)

        pool_ref[...] += lax.dot_general(
            oh, h, (((0,), (0,)), ((), ())), preferred_element_type=jnp.float32)

    return pl.pallas_call(
        body,
        grid=(nb,),
        in_specs=[
            _rows(d),
            pl.BlockSpec((BLK, d), lambda i: (p1off + i, 0)),
            _rows(d),
            pl.BlockSpec((1, 1, BLK), lambda i: (i, 0, 0)),
            _full((d, 2 * d)), _full((1, 2 * d)), _full((1, 2 * d)), _full((1, 2 * d)),
            _full((2 * d, d)), _full((1, d)),
            _full((1, d)), _full((1, d)),
        ],
        out_specs=[_rows(d), _full((NGRAPH, d))],
        out_shape=[
            jax.ShapeDtypeStruct((npad, d), jnp.float32),
            jax.ShapeDtypeStruct((NGRAPH, d), jnp.float32),
        ],
    )


@functools.lru_cache(maxsize=None)
def _make_tcv(npad, d):
    nb = npad // BLK

    def body(post_ref, b_ref, pool_ref, ve, wa, ba, ga, bea, wb, bb2, gb, beb,
             o_ref, vf_ref):
        i = pl.program_id(0)

        @pl.when(i == 0)
        def _():
            h = pool_ref[...] + ve[...]
            a = jnp.dot(h, wa[...], preferred_element_type=jnp.float32) + ba[...]
            a = jnp.maximum(a * ga[...] + bea[...], 0.0)
            v = jnp.dot(a, wb[...], preferred_element_type=jnp.float32) + bb2[...]
            vf_ref[...] = jnp.maximum(v * gb[...] + beb[...], 0.0)

        oh = _onehot(b_ref)
        o_ref[...] = post_ref[...] + jnp.dot(
            oh, vf_ref[...], preferred_element_type=jnp.float32)

    return pl.pallas_call(
        body,
        grid=(nb,),
        in_specs=[
            _rows(d),
            pl.BlockSpec((1, 1, BLK), lambda i: (i, 0, 0)),
            _full((NGRAPH, d)), _full((1, d)),
            _full((d, 2 * d)), _full((1, 2 * d)), _full((1, 2 * d)), _full((1, 2 * d)),
            _full((2 * d, d)), _full((1, d)), _full((1, d)), _full((1, d)),
        ],
        out_specs=_rows(d),
        out_shape=jax.ShapeDtypeStruct((npad, d), jnp.float32),
        scratch_shapes=[pltpu.VMEM((NGRAPH, d), jnp.float32)],
    )


@functools.lru_cache(maxsize=None)
def _make_tc3(npad, d, ncls):
    nb = npad // BLK
    p1off = npad // BLK

    def body(p_ref, q_ref, g_ref, b_ref, w1, b1, g1, be1, w2, b2, bg, bb,
             wc, bc, o_ref, sum_scr, cnt_scr):
        i = pl.program_id(0)
        hin = p_ref[...] + q_ref[...] - g_ref[...]
        h = _mlp(hin, w1[...], b1[...], g1[...], be1[...], w2[...], b2[...])
        h = h * bg[...] + bb[...]
        oh = _onehot(b_ref)

        @pl.when(i == 0)
        def _():
            sum_scr[...] = jnp.zeros_like(sum_scr)
            cnt_scr[...] = jnp.zeros_like(cnt_scr)

        sum_scr[...] += lax.dot_general(
            oh, h, (((0,), (0,)), ((), ())), preferred_element_type=jnp.float32)
        cnt_scr[...] += jnp.sum(oh, axis=0).reshape(NGRAPH, 1)

        @pl.when(i == nb - 1)
        def _():
            readout = sum_scr[...] / jnp.maximum(cnt_scr[...], 1.0)
            o_ref[...] = jnp.dot(
                readout, wc[...], preferred_element_type=jnp.float32) + bc[...]

    return pl.pallas_call(
        body,
        grid=(nb,),
        in_specs=[
            _rows(d),
            pl.BlockSpec((BLK, d), lambda i: (p1off + i, 0)),
            _rows(d),
            pl.BlockSpec((1, 1, BLK), lambda i: (i, 0, 0)),
            _full((d, 2 * d)), _full((1, 2 * d)), _full((1, 2 * d)), _full((1, 2 * d)),
            _full((2 * d, d)), _full((1, d)),
            _full((1, d)), _full((1, d)),
            _full((d, ncls)), _full((1, ncls)),
        ],
        out_specs=_full((NGRAPH, ncls)),
        out_shape=jax.ShapeDtypeStruct((NGRAPH, ncls), jnp.float32),
        scratch_shapes=[
            pltpu.VMEM((NGRAPH, d), jnp.float32),
            pltpu.VMEM((NGRAPH, 1), jnp.float32),
        ],
    )


def _r2(v, w):
    return v.reshape(1, w)


def kernel(x, params, edge_index, batch):
    n, d = x.shape
    e = edge_index.shape[1]
    ncls = params["cls"]["Wc"].shape[1]

    npad = ((n + BLK) // BLK) * BLK
    ept = -(-e // (NTILES * 2 * CH)) * 2 * CH
    epad = NTILES * ept

    x_pad = jnp.zeros((npad, d), jnp.float32).at[:n].set(x)
    src_pad = jnp.concatenate([edge_index[0], jnp.zeros((epad - e,), jnp.int32)])
    dst_pad = jnp.concatenate([edge_index[1], jnp.full((epad - e,), n, jnp.int32)])
    batch3 = jnp.concatenate(
        [batch, jnp.full((npad - n,), NGRAPH, jnp.int32)]).reshape(npad // BLK, 1, BLK)

    sc_agg = _make_sc_agg(npad, d, ept)
    c1 = params["conv1"]
    bn1 = params["bn1"]
    vm = params["vmlp"]
    ve = params["vemb"]

    p = sc_agg(x_pad, src_pad, dst_pad)
    g2 = _make_tc1(npad, d)(
        p, p, x_pad,
        c1["W1"], _r2(c1["b1"], 2 * d), _r2(c1["g1"], 2 * d), _r2(c1["be1"], 2 * d),
        c1["W2"], _r2(c1["b2"], d),
        _r2(bn1["g"], d), _r2(bn1["b"], d), ve,
    )

    c2 = params["convs"][0]
    bn2 = params["bns"][0]
    p = sc_agg(g2, src_pad, dst_pad)
    post2, pooled = _make_tc2(npad, d)(
        p, p, g2, batch3,
        c2["W1"], _r2(c2["b1"], 2 * d), _r2(c2["g1"], 2 * d), _r2(c2["be1"], 2 * d),
        c2["W2"], _r2(c2["b2"], d),
        _r2(bn2["g"], d), _r2(bn2["b"], d),
    )

    g3 = _make_tcv(npad, d)(
        post2, batch3, pooled, ve,
        vm["Wa"], _r2(vm["ba"], 2 * d), _r2(vm["ga"], 2 * d), _r2(vm["bea"], 2 * d),
        vm["Wb"], _r2(vm["bb"], d), _r2(vm["gb"], d), _r2(vm["beb"], d),
    )

    c3 = params["convs"][1]
    bn3 = params["bns"][1]
    p = sc_agg(g3, src_pad, dst_pad)
    logits = _make_tc3(npad, d, ncls)(
        p, p, g3, batch3,
        c3["W1"], _r2(c3["b1"], 2 * d), _r2(c3["g1"], 2 * d), _r2(c3["be1"], 2 * d),
        c3["W2"], _r2(c3["b2"], d),
        _r2(bn3["g"], d), _r2(bn3["b"], d),
        params["cls"]["Wc"], _r2(params["cls"]["bc"], ncls),
    )
    return logits


# speedup vs baseline: 9.7861x; 2.7647x over previous
import functools

import jax
import jax.numpy as jnp
from jax import lax
from jax.experimental import pallas as pl
from jax.experimental.pallas import tpu as pltpu
from jax.experimental.pallas import tpu_sc as plsc

BLK = 256
NGRAPH = 64
CH = 128
NTILES = 32


@functools.lru_cache(maxsize=None)
def _make_sc_agg(npad, d, ept):
    rows_per_tile = npad // 16
    nch = ept // CH
    assert nch % 2 == 0
    mesh = plsc.VectorSubcoreMesh(core_axis_name="c", subcore_axis_name="s")

    @functools.partial(
        pl.kernel,
        out_type=jax.ShapeDtypeStruct((2 * npad, d), jnp.float32),
        mesh=mesh,
        scratch_types=[
            pltpu.VMEM((ept,), jnp.int32),
            pltpu.VMEM((CH,), jnp.int32),
            pltpu.VMEM((CH,), jnp.int32),
            pltpu.VMEM((CH, d), jnp.float32),
            pltpu.VMEM((CH, d), jnp.float32),
            pltpu.VMEM_SHARED((npad, d), jnp.float32),
            pltpu.SemaphoreType.DMA,
            pltpu.SemaphoreType.DMA,
            pltpu.SemaphoreType.DMA,
            pltpu.SemaphoreType.DMA,
        ],
    )
    def sc_agg(g_hbm, src_hbm, dst_hbm, out_hbm, src_st, dst0, dst1,
               rows0, rows1, acc_sh, semg0, semg1, semi0, semi1):
        c = lax.axis_index("c")
        s = lax.axis_index("s")
        tid = s * 2 + c
        ebase = tid * ept
        pltpu.sync_copy(
            g_hbm.at[pl.ds(s * rows_per_tile, rows_per_tile)],
            acc_sh.at[pl.ds(s * rows_per_tile, rows_per_tile)],
        )
        pltpu.sync_copy(src_hbm.at[pl.ds(ebase, ept)], src_st)
        plsc.subcore_barrier()

        def fetch(j, dst_v, rows, semg, semi):
            pltpu.async_copy(dst_hbm.at[pl.ds(ebase + j * CH, CH)], dst_v, semi)
            pltpu.async_copy(g_hbm.at[src_st.at[pl.ds(j * CH, CH)]], rows, semg)

        def step(j, dst_v, rows, semg, semi, prefetch):
            pltpu.make_async_copy(
                g_hbm.at[src_st.at[pl.ds(j * CH, CH)]], rows, semg).wait()
            pltpu.make_async_copy(dst_hbm.at[pl.ds(0, CH)], dst_v, semi).wait()
            pltpu.sync_copy(rows, acc_sh.at[dst_v], add=True)
            if prefetch:
                fetch(j + 2, dst_v, rows, semg, semi)

        fetch(0, dst0, rows0, semg0, semi0)
        fetch(1, dst1, rows1, semg1, semi1)

        def body(i, carry):
            j = 2 * i
            step(j, dst0, rows0, semg0, semi0, True)
            step(j + 1, dst1, rows1, semg1, semi1, True)
            return carry

        lax.fori_loop(0, nch // 2 - 1, body, 0)
        step(nch - 2, dst0, rows0, semg0, semi0, False)
        step(nch - 1, dst1, rows1, semg1, semi1, False)

        plsc.subcore_barrier()
        pltpu.sync_copy(
            acc_sh.at[pl.ds(s * rows_per_tile, rows_per_tile)],
            out_hbm.at[pl.ds(c * npad + s * rows_per_tile, rows_per_tile)],
        )

    return sc_agg


def _mlp(hin, w1, b1, g1, be1, w2, b2):
    a = jnp.dot(hin, w1, preferred_element_type=jnp.float32) + b1
    a = jnp.maximum(a * g1 + be1, 0.0)
    return jnp.dot(a, w2, preferred_element_type=jnp.float32) + b2


def _onehot(b_ref):
    bidx = b_ref[0, 0, :].reshape(BLK, 1)
    io = lax.broadcasted_iota(jnp.int32, (BLK, NGRAPH), 1)
    return (bidx == io).astype(jnp.float32)


def _full(shape):
    nd = len(shape)
    return pl.BlockSpec(shape, lambda i: (0,) * nd)


def _rows(d):
    return pl.BlockSpec((BLK, d), lambda i: (i, 0))


@functools.lru_cache(maxsize=None)
def _make_tc1(npad, d):
    nb = npad // BLK
    p1off = npad // BLK

    def body(p_ref, q_ref, g_ref, w1, b1, g1, be1, w2, b2, bg, bb, ve, o_ref):
        hin = p_ref[...] + q_ref[...] - g_ref[...]
        h = _mlp(hin, w1[...], b1[...], g1[...], be1[...], w2[...], b2[...])
        h = jnp.maximum(h * bg[...] + bb[...], 0.0)
        o_ref[...] = h + ve[...]

    return pl.pallas_call(
        body,
        grid=(nb,),
        in_specs=[
            _rows(d),
            pl.BlockSpec((BLK, d), lambda i: (p1off + i, 0)),
            _rows(d),
            _full((d, 2 * d)), _full((1, 2 * d)), _full((1, 2 * d)), _full((1, 2 * d)),
            _full((2 * d, d)), _full((1, d)),
            _full((1, d)), _full((1, d)), _full((1, d)),
        ],
        out_specs=_rows(d),
        out_shape=jax.ShapeDtypeStruct((npad, d), jnp.float32),
    )


@functools.lru_cache(maxsize=None)
def _make_tc2(npad, d):
    nb = npad // BLK
    p1off = npad // BLK

    def body(p_ref, q_ref, g_ref, b_ref, w1, b1, g1, be1, w2, b2, bg, bb,
             post_ref, pool_ref):
        i = pl.program_id(0)
        hin = p_ref[...] + q_ref[...] - g_ref[...]
        h = _mlp(hin, w1[...], b1[...], g1[...], be1[...], w2[...], b2[...])
        h = jnp.maximum(h * bg[...] + bb[...], 0.0)
        post_ref[...] = h
        oh = _onehot(b_ref)

        @pl.when(i == 0)
        def _():
            pool_ref[...] = jnp.zeros_like(pool_ref)

        pool_ref[...] += lax.dot_general(
            oh, h, (((0,), (0,)), ((), ())), preferred_element_type=jnp.float32)

    return pl.pallas_call(
        body,
        grid=(nb,),
        in_specs=[
            _rows(d),
            pl.BlockSpec((BLK, d), lambda i: (p1off + i, 0)),
            _rows(d),
            pl.BlockSpec((1, 1, BLK), lambda i: (i, 0, 0)),
            _full((d, 2 * d)), _full((1, 2 * d)), _full((1, 2 * d)), _full((1, 2 * d)),
            _full((2 * d, d)), _full((1, d)),
            _full((1, d)), _full((1, d)),
        ],
        out_specs=[_rows(d), _full((NGRAPH, d))],
        out_shape=[
            jax.ShapeDtypeStruct((npad, d), jnp.float32),
            jax.ShapeDtypeStruct((NGRAPH, d), jnp.float32),
        ],
    )


@functools.lru_cache(maxsize=None)
def _make_tcv(npad, d):
    def body(post_ref, b_ref, pool_ref, ve, wa, ba, ga, bea, wb, bb2, gb, beb,
             o_ref, vf_ref):
        i = pl.program_id(0)

        @pl.when(i == 0)
        def _():
            h = pool_ref[...] + ve[...]
            a = jnp.dot(h, wa[...], preferred_element_type=jnp.float32) + ba[...]
            a = jnp.maximum(a * ga[...] + bea[...], 0.0)
            v = jnp.dot(a, wb[...], preferred_element_type=jnp.float32) + bb2[...]
            vf_ref[...] = jnp.maximum(v * gb[...] + beb[...], 0.0)

        oh = _onehot(b_ref)
        o_ref[...] = post_ref[...] + jnp.dot(
            oh, vf_ref[...], preferred_element_type=jnp.float32)

    return pl.pallas_call(
        body,
        grid=(npad // BLK,),
        in_specs=[
            _rows(d),
            pl.BlockSpec((1, 1, BLK), lambda i: (i, 0, 0)),
            _full((NGRAPH, d)), _full((1, d)),
            _full((d, 2 * d)), _full((1, 2 * d)), _full((1, 2 * d)), _full((1, 2 * d)),
            _full((2 * d, d)), _full((1, d)), _full((1, d)), _full((1, d)),
        ],
        out_specs=_rows(d),
        out_shape=jax.ShapeDtypeStruct((npad, d), jnp.float32),
        scratch_shapes=[pltpu.VMEM((NGRAPH, d), jnp.float32)],
    )


@functools.lru_cache(maxsize=None)
def _make_tc3(npad, d, ncls):
    nb = npad // BLK
    p1off = npad // BLK

    def body(p_ref, q_ref, g_ref, b_ref, w1, b1, g1, be1, w2, b2, bg, bb,
             wc, bc, o_ref, sum_scr, cnt_scr):
        i = pl.program_id(0)
        hin = p_ref[...] + q_ref[...] - g_ref[...]
        h = _mlp(hin, w1[...], b1[...], g1[...], be1[...], w2[...], b2[...])
        h = h * bg[...] + bb[...]
        oh = _onehot(b_ref)

        @pl.when(i == 0)
        def _():
            sum_scr[...] = jnp.zeros_like(sum_scr)
            cnt_scr[...] = jnp.zeros_like(cnt_scr)

        sum_scr[...] += lax.dot_general(
            oh, h, (((0,), (0,)), ((), ())), preferred_element_type=jnp.float32)
        cnt_scr[...] += jnp.sum(oh, axis=0).reshape(NGRAPH, 1)

        @pl.when(i == nb - 1)
        def _():
            readout = sum_scr[...] / jnp.maximum(cnt_scr[...], 1.0)
            o_ref[...] = jnp.dot(
                readout, wc[...], preferred_element_type=jnp.float32) + bc[...]

    return pl.pallas_call(
        body,
        grid=(nb,),
        in_specs=[
            _rows(d),
            pl.BlockSpec((BLK, d), lambda i: (p1off + i, 0)),
            _rows(d),
            pl.BlockSpec((1, 1, BLK), lambda i: (i, 0, 0)),
            _full((d, 2 * d)), _full((1, 2 * d)), _full((1, 2 * d)), _full((1, 2 * d)),
            _full((2 * d, d)), _full((1, d)),
            _full((1, d)), _full((1, d)),
            _full((d, ncls)), _full((1, ncls)),
        ],
        out_specs=_full((NGRAPH, ncls)),
        out_shape=jax.ShapeDtypeStruct((NGRAPH, ncls), jnp.float32),
        scratch_shapes=[
            pltpu.VMEM((NGRAPH, d), jnp.float32),
            pltpu.VMEM((NGRAPH, 1), jnp.float32),
        ],
    )


def _r2(v, w):
    return v.reshape(1, w)


def kernel(x, params, edge_index, batch):
    n, d = x.shape
    e = edge_index.shape[1]
    ncls = params["cls"]["Wc"].shape[1]

    npad = ((n + BLK) // BLK) * BLK
    ept = -(-e // (NTILES * 2 * CH)) * 2 * CH
    epad = NTILES * ept

    x_pad = jnp.zeros((npad, d), jnp.float32).at[:n].set(x)
    padrows = jnp.arange(epad - e, dtype=jnp.int32)
    src_pad = jnp.concatenate([edge_index[0], padrows % n])
    dst_pad = jnp.concatenate([edge_index[1], n + padrows % (npad - n)])
    batch3 = jnp.concatenate(
        [batch, jnp.full((npad - n,), NGRAPH, jnp.int32)]).reshape(npad // BLK, 1, BLK)

    sc_agg = _make_sc_agg(npad, d, ept)
    c1 = params["conv1"]
    bn1 = params["bn1"]
    vm = params["vmlp"]
    ve = params["vemb"]

    p = sc_agg(x_pad, src_pad, dst_pad)
    g2 = _make_tc1(npad, d)(
        p, p, x_pad,
        c1["W1"], _r2(c1["b1"], 2 * d), _r2(c1["g1"], 2 * d), _r2(c1["be1"], 2 * d),
        c1["W2"], _r2(c1["b2"], d),
        _r2(bn1["g"], d), _r2(bn1["b"], d), ve,
    )

    c2 = params["convs"][0]
    bn2 = params["bns"][0]
    p = sc_agg(g2, src_pad, dst_pad)
    post2, pooled = _make_tc2(npad, d)(
        p, p, g2, batch3,
        c2["W1"], _r2(c2["b1"], 2 * d), _r2(c2["g1"], 2 * d), _r2(c2["be1"], 2 * d),
        c2["W2"], _r2(c2["b2"], d),
        _r2(bn2["g"], d), _r2(bn2["b"], d),
    )

    g3 = _make_tcv(npad, d)(
        post2, batch3, pooled, ve,
        vm["Wa"], _r2(vm["ba"], 2 * d), _r2(vm["ga"], 2 * d), _r2(vm["bea"], 2 * d),
        vm["Wb"], _r2(vm["bb"], d), _r2(vm["gb"], d), _r2(vm["beb"], d),
    )

    c3 = params["convs"][1]
    bn3 = params["bns"][1]
    p = sc_agg(g3, src_pad, dst_pad)
    logits = _make_tc3(npad, d, ncls)(
        p, p, g3, batch3,
        c3["W1"], _r2(c3["b1"], 2 * d), _r2(c3["g1"], 2 * d), _r2(c3["be1"], 2 * d),
        c3["W2"], _r2(c3["b2"], d),
        _r2(bn3["g"], d), _r2(bn3["b"], d),
        params["cls"]["Wc"], _r2(params["cls"]["bc"], ncls),
    )
    return logits
